# baseline (speedup 1.0000x reference)
import functools

import jax
import jax.numpy as jnp
from jax import lax
from jax.experimental import pallas as pl
from jax.experimental.pallas import tpu as pltpu
from jax.experimental.pallas import tpu_sc as plsc

_BATCH = 4096
_EMB = 16
_NU = 2
_NI = 10
_LAYER = 256
_XDIM = 15448
_TOT = _NU * _EMB + _NI * _EMB + _XDIM

_NC = 2
_NS = 16
_NW = _NC * _NS

_CHUNK = 128

_UB = _BATCH * _NU
_IB = _BATCH * _NI
_U_PER = _UB // _NW
_I_PER = _IB // _NW
_UC = _U_PER // _CHUNK
_IC = _I_PER // _CHUNK


_NB = _BATCH // _NW
_QU = _NU * _EMB
_QI = _NI * _EMB
_Q = _QU + _QI
_NCHUNK = _Q // _EMB


def _sc_gather_body(idx_all, u_flat, i_flat, e3_out, idx_v, rows_v, sem):
    wid = lax.axis_index("s") * _NC + lax.axis_index("c")
    pltpu.sync_copy(idx_all.at[wid], idx_v)

    def chunk(g, carry):
        base = g * _EMB

        @pl.when(g < _QU // _EMB)
        def _():
            cps = [pltpu.async_copy(u_flat.at[idx_v.at[base + j]],
                                    rows_v.at[base + j], sem)
                   for j in range(_EMB)]
            for cp in cps:
                cp.wait()

        @pl.when(g >= _QU // _EMB)
        def _():
            cps = [pltpu.async_copy(i_flat.at[idx_v.at[base + j]],
                                    rows_v.at[base + j], sem)
                   for j in range(_EMB)]
            for cp in cps:
                cp.wait()

        return carry

    lax.fori_loop(0, _NCHUNK, chunk, 0)
    pltpu.sync_copy(rows_v, e3_out.at[wid])


@functools.lru_cache(maxsize=1)
def _sc_gather():
    return pl.kernel(
        _sc_gather_body,
        mesh=plsc.VectorSubcoreMesh(core_axis_name="c", subcore_axis_name="s"),
        out_type=jax.ShapeDtypeStruct((_NW, _Q, _NB), jnp.float32),
        scratch_types=[
            pltpu.VMEM((_Q, _NB), jnp.int32),
            pltpu.VMEM((_Q, _NB), jnp.float32),
            pltpu.SemaphoreType.DMA,
        ],
        compiler_params=pltpu.CompilerParams(use_tc_tiling_on_sc=False),
    )


_TK = 1024
_KT = 16
_KLAST = _XDIM - (_KT - 1) * _TK


def _p1_body(xt_ref, w1x_ref, h1_ref):
    k = pl.program_id(0)

    def _acc(p):
        @pl.when(k == 0)
        def _():
            h1_ref[...] = p

        @pl.when(k != 0)
        def _():
            h1_ref[...] += p

    @pl.when(k != _KT - 1)
    def _():
        _acc(lax.dot_general(
            xt_ref[...], w1x_ref[...],
            dimension_numbers=(((0,), (0,)), ((), ())),
            preferred_element_type=jnp.float32))

    @pl.when(k == _KT - 1)
    def _():
        valid = lax.broadcasted_iota(jnp.int32, (_TK, 1), 0) < _KLAST
        xm = jnp.where(valid, xt_ref[...], 0.0)
        wm = jnp.where(valid, w1x_ref[...], 0.0)
        _acc(lax.dot_general(
            xm, wm,
            dimension_numbers=(((0,), (0,)), ((), ())),
            preferred_element_type=jnp.float32))


def _p1(xt, W1x):
    return pl.pallas_call(
        _p1_body,
        grid=(_KT,),
        in_specs=[
            pl.BlockSpec((_TK, _BATCH), lambda k: (k, 0)),
            pl.BlockSpec((_TK, _LAYER), lambda k: (k, 0)),
        ],
        out_specs=pl.BlockSpec((_BATCH, _LAYER), lambda k: (0, 0)),
        out_shape=jax.ShapeDtypeStruct((_BATCH, _LAYER), jnp.float32),
        compiler_params=pltpu.CompilerParams(
            vmem_limit_bytes=100 * 1024 * 1024),
    )(xt, W1x)


def _p2_body(h1x_ref, e3_ref, w1e_ref, g1_ref, be1_ref,
             w2_ref, g2_ref, be2_ref, w3_ref, b3_ref, out_ref, h1_scr):
    for w in range(_NW):
        emb = lax.dot_general(
            e3_ref[w], w1e_ref[...],
            dimension_numbers=(((0,), (0,)), ((), ())),
            preferred_element_type=jnp.float32)
        h1_scr[pl.ds(w * _NB, _NB), :] = (
            h1x_ref[pl.ds(w * _NB, _NB), :] + emb)
    h1 = h1_scr[...]
    m1 = jnp.mean(h1, axis=0, keepdims=True)
    v1 = jnp.mean((h1 - m1) * (h1 - m1), axis=0, keepdims=True)
    h = (h1 - m1) * lax.rsqrt(v1 + 1e-5) * g1_ref[...] + be1_ref[...]
    h = jnp.maximum(h, 0.0)
    h2 = jnp.dot(h, w2_ref[...], preferred_element_type=jnp.float32)
    m2 = jnp.mean(h2, axis=0, keepdims=True)
    v2 = jnp.mean((h2 - m2) * (h2 - m2), axis=0, keepdims=True)
    h2 = (h2 - m2) * lax.rsqrt(v2 + 1e-5) * g2_ref[...] + be2_ref[...]
    h2 = jnp.maximum(h2, 0.0)
    out_ref[...] = (jnp.dot(h2, w3_ref[...],
                            preferred_element_type=jnp.float32)
                    + b3_ref[...])


def _p2(h1x, e3, W1e, g1, be1, W2, g2, be2, W3, b3):
    full = lambda s: pl.BlockSpec(s, lambda: (0,) * len(s))
    return pl.pallas_call(
        _p2_body,
        in_specs=[
            full((_BATCH, _LAYER)),
            full((_NW, _Q, _NB)),
            full((_Q, _LAYER)),
            full((1, _LAYER)),
            full((1, _LAYER)),
            full((_LAYER, _LAYER)),
            full((1, _LAYER)),
            full((1, _LAYER)),
            full((_LAYER, _NI)),
            full((1, _NI)),
        ],
        out_specs=full((_BATCH, _NI)),
        out_shape=jax.ShapeDtypeStruct((_BATCH, _NI), jnp.float32),
        scratch_shapes=[pltpu.VMEM((_BATCH, _LAYER), jnp.float32)],
        compiler_params=pltpu.CompilerParams(
            vmem_limit_bytes=100 * 1024 * 1024),
    )(h1x, e3, W1e, g1, be1, W2, g2, be2, W3, b3)


def kernel(x, u_cats, i_cats, user_table, item_table,
           W1, b1, g1, be1, W2, b2, g2, be2, W3, b3):
    u_flat = user_table.T.reshape(_EMB * 1000000)
    i_flat = item_table.T.reshape(_EMB * 1000000)
    dim_off = (jnp.arange(_EMB, dtype=jnp.int32) * 1000000)
    uc = u_cats.reshape(_NW, _NB, _NU).transpose(0, 2, 1)
    ui = (uc[:, :, None, :] + dim_off[None, None, :, None]
          ).reshape(_NW, _QU, _NB)
    ic = i_cats.reshape(_NW, _NB, _NI).transpose(0, 2, 1)
    ii = (ic[:, :, None, :] + dim_off[None, None, :, None]
          ).reshape(_NW, _QI, _NB)
    idx_all = jnp.concatenate([ui, ii], axis=1)
    e3 = _sc_gather()(idx_all, u_flat, i_flat)
    nde = _NU * _EMB + _NI * _EMB
    h1x = _p1(x.T, W1[nde:, :])
    return _p2(h1x, e3, W1[:nde, :],
               g1.reshape(1, _LAYER), be1.reshape(1, _LAYER),
               W2, g2.reshape(1, _LAYER), be2.reshape(1, _LAYER),
               W3, b3.reshape(1, _NI))
